# baseline (speedup 1.0000x reference)
import functools

import jax
import jax.numpy as jnp
from jax import lax
from jax.experimental import pallas as pl
from jax.experimental.pallas import tpu as pltpu
from jax.experimental.pallas import tpu_sc as plsc

_N = 10000
_E = 320000
_D_IN = 128
_D_H = 256
_D_EDGE = 16
_G = 64

_NC = 2
_NS = 16
_L = 16

_SLAB = 632
_ACC_R = 640
_DUMP = _SLAB
_SCAN = 2000
_NSC = _E // _SCAN
_F = 64

_BN = 1000
_BE = 4000


def _sc_agg(hw2, ea2, src, dst, zacc):
  mesh = plsc.VectorSubcoreMesh(core_axis_name="c", subcore_axis_name="s",
                                num_cores=_NC, num_subcores=_NS)

  @functools.partial(
      pl.kernel,
      mesh=mesh,
      compiler_params=pltpu.CompilerParams(needs_layout_passes=False),
      out_type=jax.ShapeDtypeStruct((_NC, _NS, _ACC_R, 128), jnp.float32),
      scratch_types=[
          pltpu.VMEM((_ACC_R, 128), jnp.float32),
          pltpu.VMEM((_SCAN,), jnp.int32),
          pltpu.VMEM((_SCAN,), jnp.int32),
          pltpu.VMEM((2 * _F,), jnp.int32),
          pltpu.VMEM((2 * _F,), jnp.int32),
          pltpu.VMEM((2 * _F,), jnp.int32),
          pltpu.VMEM((_F, 128), jnp.float32),
          pltpu.VMEM((_F, 128), jnp.float32),
          pltpu.SemaphoreType.DMA,
          pltpu.SemaphoreType.DMA,
      ],
  )
  def seg_kernel(hw_hbm, ea_hbm, src_hbm, dst_hbm, z_hbm, out_hbm, acc,
                 dbuf, sbuf, gbuf, ebuf, lbuf, rows, earows, sem, sem2):
    c = lax.axis_index("c")
    s = lax.axis_index("s")
    lo = s * _SLAB
    goff = c * _N
    eoff = c * _E
    pltpu.sync_copy(z_hbm, acc)

    def flush(off):
      pltpu.async_copy(hw_hbm.at[gbuf.at[pl.ds(off, _F)]], rows, sem)
      pltpu.async_copy(ea_hbm.at[ebuf.at[pl.ds(off, _F)]], earows, sem2)
      pltpu.make_async_copy(hw_hbm.at[gbuf.at[pl.ds(off, _F)]], rows,
                            sem).wait()
      pltpu.make_async_copy(ea_hbm.at[ebuf.at[pl.ds(off, _F)]], earows,
                            sem2).wait()
      for k in range(_F // _L):
        lv = lbuf[pl.ds(off + k * _L, _L)]
        for t in range(_L):
          e = k * _L + t
          d = lv[t]
          for q in range(8):
            sl = pl.ds(q * _L, _L)
            plsc.addupdate(acc.at[d, sl], rows[e, sl] + earows[e, sl])

    def scan_chunk(j, cursor):
      pltpu.sync_copy(dst_hbm.at[pl.ds(j * _SCAN, _SCAN)], dbuf)
      pltpu.sync_copy(src_hbm.at[pl.ds(j * _SCAN, _SCAN)], sbuf)

      def step(i, cursor):
        dv = dbuf[pl.ds(i * _L, _L)]
        sv = sbuf[pl.ds(i * _L, _L)]
        m = (dv >= lo) & (dv < lo + _SLAB)
        eid = lax.iota(jnp.int32, _L) + (j * _SCAN + i * _L + eoff)
        mi = jnp.where(m, 1, 0)
        ps = plsc.cumsum(mi)
        pos = (cursor + ps - 1) * mi + (2 * _F - 1) * (1 - mi)
        plsc.store_scatter(gbuf, [pos], sv + goff)
        plsc.store_scatter(ebuf, [pos], eid)
        plsc.store_scatter(lbuf, [pos], dv - lo)
        cursor = cursor + ps[_L - 1]

        @pl.when(cursor >= _F)
        def _():
          flush(0)
          gt = gbuf[pl.ds(_F, _L)]
          et = ebuf[pl.ds(_F, _L)]
          lt = lbuf[pl.ds(_F, _L)]
          gbuf[pl.ds(0, _L)] = gt
          ebuf[pl.ds(0, _L)] = et
          lbuf[pl.ds(0, _L)] = lt

        cursor = jnp.where(cursor >= _F, cursor - _F, cursor)
        return cursor

      return lax.fori_loop(0, _SCAN // _L, step, cursor, unroll=False)

    cursor = lax.fori_loop(0, _NSC, scan_chunk, jnp.int32(0), unroll=False)

    iot = lax.iota(jnp.int32, _L)
    for k in range(_F // _L):
      sl = pl.ds(k * _L, _L)
      keep = (iot + (k * _L)) < cursor
      gbuf[sl] = jnp.where(keep, gbuf[sl], goff)
      ebuf[sl] = jnp.where(keep, ebuf[sl], eoff)
      lbuf[sl] = jnp.where(keep, lbuf[sl], _DUMP)
    flush(0)

    pltpu.sync_copy(acc, out_hbm.at[c, s])

  return seg_kernel(hw2, ea2, src, dst, zacc)


def _tc_emb(x, W_emb, b_emb2, W_nbr):
  nb = _N // _BN

  def body(x_ref, w_ref, b_ref, wn_ref, h_ref, hw_ref):
    h = jnp.dot(x_ref[...], w_ref[...], preferred_element_type=jnp.float32)
    h = jnp.maximum(h + b_ref[...], 0.0)
    h_ref[0] = h[:, :128]
    h_ref[1] = h[:, 128:]
    hw = jnp.dot(h, wn_ref[...], preferred_element_type=jnp.float32)
    hw_ref[0] = hw[:, :128]
    hw_ref[1] = hw[:, 128:]

  return pl.pallas_call(
      body,
      grid=(nb,),
      in_specs=[
          pl.BlockSpec((_BN, _D_IN), lambda i: (i, 0)),
          pl.BlockSpec((_D_IN, _D_H), lambda i: (0, 0)),
          pl.BlockSpec((1, _D_H), lambda i: (0, 0)),
          pl.BlockSpec((_D_H, _D_H), lambda i: (0, 0)),
      ],
      out_specs=[
          pl.BlockSpec((2, _BN, 128), lambda i: (0, i, 0)),
          pl.BlockSpec((2, _BN, 128), lambda i: (0, i, 0)),
      ],
      out_shape=[
          jax.ShapeDtypeStruct((2, _N, 128), jnp.float32),
          jax.ShapeDtypeStruct((2, _N, 128), jnp.float32),
      ],
  )(x, W_emb, b_emb2, W_nbr)


def _tc_eaw(edge_attr, W_edge):
  nb = _E // _BE

  def body(ea_ref, we_ref, out_ref):
    eaw = jnp.dot(ea_ref[...], we_ref[...], preferred_element_type=jnp.float32)
    out_ref[0] = eaw[:, :128]
    out_ref[1] = eaw[:, 128:]

  return pl.pallas_call(
      body,
      grid=(nb,),
      in_specs=[
          pl.BlockSpec((_BE, _D_EDGE), lambda i: (i, 0)),
          pl.BlockSpec((_D_EDGE, _D_H), lambda i: (0, 0)),
      ],
      out_specs=pl.BlockSpec((2, _BE, 128), lambda i: (0, i, 0)),
      out_shape=jax.ShapeDtypeStruct((2, _E, 128), jnp.float32),
  )(edge_attr, W_edge)


def _tc_conv(h_split, AGG, batch2, W_self, b_conv2, W_gate, b_gate2):
  nb = _N // _BN

  def body(h_ref, agg_ref, b_ref, ws_ref, bc_ref, wg_ref, bg_ref, h2_ref,
           gmax_ref):
    i = pl.program_id(0)
    z = jnp.dot(h_ref[0], ws_ref[:128], preferred_element_type=jnp.float32)
    z += jnp.dot(h_ref[1], ws_ref[128:], preferred_element_type=jnp.float32)
    h2 = jnp.maximum(z + agg_ref[...] + bc_ref[...], 0.0)
    h2_ref[...] = h2
    gate = jnp.dot(h2, wg_ref[...], preferred_element_type=jnp.float32)
    gate += bg_ref[...]
    mask = lax.broadcasted_iota(jnp.int32, (_BN, _G), 1) == b_ref[...]
    gm = jnp.max(jnp.where(mask, gate, -jnp.inf), axis=0, keepdims=True)

    @pl.when(i == 0)
    def _():
      gmax_ref[...] = gm

    @pl.when(i > 0)
    def _():
      gmax_ref[...] = jnp.maximum(gmax_ref[...], gm)

  return pl.pallas_call(
      body,
      grid=(nb,),
      in_specs=[
          pl.BlockSpec((2, _BN, 128), lambda i: (0, i, 0)),
          pl.BlockSpec((_BN, _D_H), lambda i: (i, 0)),
          pl.BlockSpec((_BN, 1), lambda i: (i, 0)),
          pl.BlockSpec((_D_H, _D_H), lambda i: (0, 0)),
          pl.BlockSpec((1, _D_H), lambda i: (0, 0)),
          pl.BlockSpec((_D_H, 1), lambda i: (0, 0)),
          pl.BlockSpec((1, 1), lambda i: (0, 0)),
      ],
      out_specs=[
          pl.BlockSpec((_BN, _D_H), lambda i: (i, 0)),
          pl.BlockSpec((1, _G), lambda i: (0, 0)),
      ],
      out_shape=[
          jax.ShapeDtypeStruct((_N, _D_H), jnp.float32),
          jax.ShapeDtypeStruct((1, _G), jnp.float32),
      ],
  )(h_split, AGG, batch2, W_self, b_conv2, W_gate, b_gate2)


def _tc_pool(h2, batch2, gmax, W_gate, b_gate2, W_p1, b_p12, W_p2, b_p22):
  nb = _N // _BN

  def body(h2_ref, b_ref, gm_ref, wg_ref, bg_ref, wp1_ref, bp1_ref, wp2_ref,
           bp2_ref, out_ref, up_acc, den_acc):
    i = pl.program_id(0)

    @pl.when(i == 0)
    def _():
      up_acc[...] = jnp.zeros_like(up_acc)
      den_acc[...] = jnp.zeros_like(den_acc)

    h2 = h2_ref[...]
    gate = jnp.dot(h2, wg_ref[...], preferred_element_type=jnp.float32)
    gate += bg_ref[...]
    mask = lax.broadcasted_iota(jnp.int32, (_BN, _G), 1) == b_ref[...]
    gm_row = jnp.sum(jnp.where(mask, gm_ref[...], 0.0), axis=1, keepdims=True)
    e = jnp.exp(gate - gm_row)
    we = jnp.where(mask, e, 0.0)
    den_acc[...] += lax.dot_general(we, jnp.ones((_BN, 1), jnp.float32),
                                    (((0,), (0,)), ((), ())),
                                    preferred_element_type=jnp.float32)
    up_acc[...] += lax.dot_general(we, h2, (((0,), (0,)), ((), ())),
                                   preferred_element_type=jnp.float32)

    @pl.when(i == nb - 1)
    def _():
      den = den_acc[...]
      pooled = up_acc[...] * jnp.where(den > 0.5, 1.0 / den, 0.0)
      p = jnp.dot(pooled, wp1_ref[...], preferred_element_type=jnp.float32)
      p = jnp.maximum(p + bp1_ref[...], 0.0)
      o = jnp.dot(p, wp2_ref[...], preferred_element_type=jnp.float32)
      out_ref[...] = o + bp2_ref[...]

  return pl.pallas_call(
      body,
      grid=(nb,),
      in_specs=[
          pl.BlockSpec((_BN, _D_H), lambda i: (i, 0)),
          pl.BlockSpec((_BN, 1), lambda i: (i, 0)),
          pl.BlockSpec((1, _G), lambda i: (0, 0)),
          pl.BlockSpec((_D_H, 1), lambda i: (0, 0)),
          pl.BlockSpec((1, 1), lambda i: (0, 0)),
          pl.BlockSpec((_D_H, 128), lambda i: (0, 0)),
          pl.BlockSpec((1, 128), lambda i: (0, 0)),
          pl.BlockSpec((128, 1), lambda i: (0, 0)),
          pl.BlockSpec((1, 1), lambda i: (0, 0)),
      ],
      out_specs=pl.BlockSpec((_G, 1), lambda i: (0, 0)),
      out_shape=jax.ShapeDtypeStruct((_G, 1), jnp.float32),
      scratch_shapes=[
          pltpu.VMEM((_G, _D_H), jnp.float32),
          pltpu.VMEM((_G, 1), jnp.float32),
      ],
  )(h2, batch2, gmax, W_gate, b_gate2, W_p1, b_p12, W_p2, b_p22)


def kernel(x, edge_index, edge_attr, batch, W_emb, b_emb, W_self, W_nbr,
           W_edge, b_conv, W_gate, b_gate, W_p1, b_p1, W_p2, b_p2):
  src = edge_index[0]
  dst = edge_index[1]
  zacc = jnp.zeros((_ACC_R, 128), jnp.float32)
  batch2 = batch.reshape(_N, 1)

  h_split, hw = _tc_emb(x, W_emb, b_emb.reshape(1, _D_H), W_nbr)
  eaw = _tc_eaw(edge_attr, W_edge)
  agg4 = _sc_agg(hw.reshape(2 * _N, 128), eaw.reshape(2 * _E, 128), src, dst,
                 zacc)
  AGG = (agg4[:, :, :_SLAB, :].reshape(_NC, _NS * _SLAB, 128)[:, :_N, :]
         .transpose(1, 0, 2).reshape(_N, _D_H))
  h2, gmax = _tc_conv(h_split, AGG, batch2, W_self, b_conv.reshape(1, _D_H),
                      W_gate, b_gate.reshape(1, 1))
  out = _tc_pool(h2, batch2, gmax, W_gate, b_gate.reshape(1, 1), W_p1,
                 b_p1.reshape(1, 128), W_p2, b_p2.reshape(1, 1))
  return out[:, 0]


# speedup vs baseline: 1.2760x; 1.2760x over previous
import functools

import jax
import jax.numpy as jnp
from jax import lax
from jax.experimental import pallas as pl
from jax.experimental.pallas import tpu as pltpu
from jax.experimental.pallas import tpu_sc as plsc

_N = 10000
_E = 320000
_D_IN = 128
_D_H = 256
_D_EDGE = 16
_G = 64

_NC = 2
_NS = 16
_L = 16

_SLAB = 632
_ACC_R = 640
_DUMP = _SLAB
_SCAN = 2000
_NSC = _E // _SCAN
_F = 128

_BN = 1000
_BE = 4000


def _sc_agg(hw2, ea2, src, dst, zacc):
  mesh = plsc.VectorSubcoreMesh(core_axis_name="c", subcore_axis_name="s",
                                num_cores=_NC, num_subcores=_NS)

  @functools.partial(
      pl.kernel,
      mesh=mesh,
      compiler_params=pltpu.CompilerParams(needs_layout_passes=False),
      out_type=jax.ShapeDtypeStruct((_NC, _NS, _ACC_R, 128), jnp.float32),
      scratch_types=[
          pltpu.VMEM((_ACC_R, 128), jnp.float32),
          pltpu.VMEM((_SCAN,), jnp.int32),
          pltpu.VMEM((_SCAN,), jnp.int32),
          pltpu.VMEM((2 * _F,), jnp.int32),
          pltpu.VMEM((2 * _F,), jnp.int32),
          pltpu.VMEM((2 * _F,), jnp.int32),
          pltpu.VMEM((_F, 128), jnp.float32),
          pltpu.VMEM((_F, 128), jnp.float32),
          pltpu.SemaphoreType.DMA,
          pltpu.SemaphoreType.DMA,
      ],
  )
  def seg_kernel(hw_hbm, ea_hbm, src_hbm, dst_hbm, z_hbm, out_hbm, acc,
                 dbuf, sbuf, gbuf, ebuf, lbuf, rows, earows, sem, sem2):
    c = lax.axis_index("c")
    s = lax.axis_index("s")
    lo = s * _SLAB
    goff = c * _N
    eoff = c * _E
    iot = lax.iota(jnp.int32, _L)
    pltpu.sync_copy(z_hbm, acc)

    def flush(off):
      pltpu.async_copy(hw_hbm.at[gbuf.at[pl.ds(off, _F)]], rows, sem)
      pltpu.async_copy(ea_hbm.at[ebuf.at[pl.ds(off, _F)]], earows, sem2)
      pltpu.make_async_copy(hw_hbm.at[gbuf.at[pl.ds(off, _F)]], rows,
                            sem).wait()
      pltpu.make_async_copy(ea_hbm.at[ebuf.at[pl.ds(off, _F)]], earows,
                            sem2).wait()
      def group(k, carry):
        lv = lbuf[pl.ds(off + k * _L, _L)]
        for t in range(_L):
          d = lv[t]
          for q in range(8):
            sl = pl.ds(q * _L, _L)
            plsc.addupdate(acc.at[d, sl],
                           rows[k * _L + t, sl] + earows[k * _L + t, sl])
        return carry

      lax.fori_loop(0, _F // _L, group, 0, unroll=False)

    def scan_chunk(j, cursor):
      pltpu.sync_copy(dst_hbm.at[pl.ds(j * _SCAN, _SCAN)], dbuf)
      pltpu.sync_copy(src_hbm.at[pl.ds(j * _SCAN, _SCAN)], sbuf)

      def step(i, cursor):
        dv = dbuf[pl.ds(i * _L, _L)]
        sv = sbuf[pl.ds(i * _L, _L)]
        m = (dv >= lo) & (dv < lo + _SLAB)
        eid = iot + (j * _SCAN + i * _L + eoff)
        mi = jnp.where(m, 1, 0)
        ps = plsc.cumsum(mi)
        pos = (cursor + ps - 1) * mi + (2 * _F - 1) * (1 - mi)
        plsc.store_scatter(gbuf, [pos], sv + goff)
        plsc.store_scatter(ebuf, [pos], eid)
        plsc.store_scatter(lbuf, [pos], dv - lo)
        cursor = cursor + ps[_L - 1]

        @pl.when(cursor >= _F)
        def _():
          flush(0)
          gt = gbuf[pl.ds(_F, _L)]
          et = ebuf[pl.ds(_F, _L)]
          lt = lbuf[pl.ds(_F, _L)]
          gbuf[pl.ds(0, _L)] = gt
          ebuf[pl.ds(0, _L)] = et
          lbuf[pl.ds(0, _L)] = lt

        cursor = jnp.where(cursor >= _F, cursor - _F, cursor)
        return cursor

      return lax.fori_loop(0, _SCAN // _L, step, cursor, unroll=False)

    cursor = lax.fori_loop(0, _NSC, scan_chunk, jnp.int32(0), unroll=False)

    for k in range(_F // _L):
      sl = pl.ds(k * _L, _L)
      keep = (iot + (k * _L)) < cursor
      gbuf[sl] = jnp.where(keep, gbuf[sl], goff)
      ebuf[sl] = jnp.where(keep, ebuf[sl], eoff)
      lbuf[sl] = jnp.where(keep, lbuf[sl], _DUMP)
    flush(0)

    pltpu.sync_copy(acc, out_hbm.at[c, s])

  return seg_kernel(hw2, ea2, src, dst, zacc)


def _tc_emb(x, W_emb, b_emb2, W_nbr):
  nb = _N // _BN

  def body(x_ref, w_ref, b_ref, wn_ref, h_ref, hw_ref):
    h = jnp.dot(x_ref[...], w_ref[...], preferred_element_type=jnp.float32)
    h = jnp.maximum(h + b_ref[...], 0.0)
    h_ref[0] = h[:, :128]
    h_ref[1] = h[:, 128:]
    hw = jnp.dot(h, wn_ref[...], preferred_element_type=jnp.float32)
    hw_ref[0] = hw[:, :128]
    hw_ref[1] = hw[:, 128:]

  return pl.pallas_call(
      body,
      grid=(nb,),
      in_specs=[
          pl.BlockSpec((_BN, _D_IN), lambda i: (i, 0)),
          pl.BlockSpec((_D_IN, _D_H), lambda i: (0, 0)),
          pl.BlockSpec((1, _D_H), lambda i: (0, 0)),
          pl.BlockSpec((_D_H, _D_H), lambda i: (0, 0)),
      ],
      out_specs=[
          pl.BlockSpec((2, _BN, 128), lambda i: (0, i, 0)),
          pl.BlockSpec((2, _BN, 128), lambda i: (0, i, 0)),
      ],
      out_shape=[
          jax.ShapeDtypeStruct((2, _N, 128), jnp.float32),
          jax.ShapeDtypeStruct((2, _N, 128), jnp.float32),
      ],
  )(x, W_emb, b_emb2, W_nbr)


def _tc_eaw(edge_attr, W_edge):
  nb = _E // _BE

  def body(ea_ref, we_ref, out_ref):
    eaw = jnp.dot(ea_ref[...], we_ref[...], preferred_element_type=jnp.float32)
    out_ref[0] = eaw[:, :128]
    out_ref[1] = eaw[:, 128:]

  return pl.pallas_call(
      body,
      grid=(nb,),
      in_specs=[
          pl.BlockSpec((_BE, _D_EDGE), lambda i: (i, 0)),
          pl.BlockSpec((_D_EDGE, _D_H), lambda i: (0, 0)),
      ],
      out_specs=pl.BlockSpec((2, _BE, 128), lambda i: (0, i, 0)),
      out_shape=jax.ShapeDtypeStruct((2, _E, 128), jnp.float32),
  )(edge_attr, W_edge)


def _tc_conv(h_split, AGG, batch2, W_self, b_conv2, W_gate, b_gate2):
  nb = _N // _BN

  def body(h_ref, agg_ref, b_ref, ws_ref, bc_ref, wg_ref, bg_ref, h2_ref,
           gmax_ref):
    i = pl.program_id(0)
    z = jnp.dot(h_ref[0], ws_ref[:128], preferred_element_type=jnp.float32)
    z += jnp.dot(h_ref[1], ws_ref[128:], preferred_element_type=jnp.float32)
    h2 = jnp.maximum(z + agg_ref[...] + bc_ref[...], 0.0)
    h2_ref[...] = h2
    gate = jnp.dot(h2, wg_ref[...], preferred_element_type=jnp.float32)
    gate += bg_ref[...]
    mask = lax.broadcasted_iota(jnp.int32, (_BN, _G), 1) == b_ref[...]
    gm = jnp.max(jnp.where(mask, gate, -jnp.inf), axis=0, keepdims=True)

    @pl.when(i == 0)
    def _():
      gmax_ref[...] = gm

    @pl.when(i > 0)
    def _():
      gmax_ref[...] = jnp.maximum(gmax_ref[...], gm)

  return pl.pallas_call(
      body,
      grid=(nb,),
      in_specs=[
          pl.BlockSpec((2, _BN, 128), lambda i: (0, i, 0)),
          pl.BlockSpec((_BN, _D_H), lambda i: (i, 0)),
          pl.BlockSpec((_BN, 1), lambda i: (i, 0)),
          pl.BlockSpec((_D_H, _D_H), lambda i: (0, 0)),
          pl.BlockSpec((1, _D_H), lambda i: (0, 0)),
          pl.BlockSpec((_D_H, 1), lambda i: (0, 0)),
          pl.BlockSpec((1, 1), lambda i: (0, 0)),
      ],
      out_specs=[
          pl.BlockSpec((_BN, _D_H), lambda i: (i, 0)),
          pl.BlockSpec((1, _G), lambda i: (0, 0)),
      ],
      out_shape=[
          jax.ShapeDtypeStruct((_N, _D_H), jnp.float32),
          jax.ShapeDtypeStruct((1, _G), jnp.float32),
      ],
  )(h_split, AGG, batch2, W_self, b_conv2, W_gate, b_gate2)


def _tc_pool(h2, batch2, gmax, W_gate, b_gate2, W_p1, b_p12, W_p2, b_p22):
  nb = _N // _BN

  def body(h2_ref, b_ref, gm_ref, wg_ref, bg_ref, wp1_ref, bp1_ref, wp2_ref,
           bp2_ref, out_ref, up_acc, den_acc):
    i = pl.program_id(0)

    @pl.when(i == 0)
    def _():
      up_acc[...] = jnp.zeros_like(up_acc)
      den_acc[...] = jnp.zeros_like(den_acc)

    h2 = h2_ref[...]
    gate = jnp.dot(h2, wg_ref[...], preferred_element_type=jnp.float32)
    gate += bg_ref[...]
    mask = lax.broadcasted_iota(jnp.int32, (_BN, _G), 1) == b_ref[...]
    gm_row = jnp.sum(jnp.where(mask, gm_ref[...], 0.0), axis=1, keepdims=True)
    e = jnp.exp(gate - gm_row)
    we = jnp.where(mask, e, 0.0)
    den_acc[...] += lax.dot_general(we, jnp.ones((_BN, 1), jnp.float32),
                                    (((0,), (0,)), ((), ())),
                                    preferred_element_type=jnp.float32)
    up_acc[...] += lax.dot_general(we, h2, (((0,), (0,)), ((), ())),
                                   preferred_element_type=jnp.float32)

    @pl.when(i == nb - 1)
    def _():
      den = den_acc[...]
      pooled = up_acc[...] * jnp.where(den > 0.5, 1.0 / den, 0.0)
      p = jnp.dot(pooled, wp1_ref[...], preferred_element_type=jnp.float32)
      p = jnp.maximum(p + bp1_ref[...], 0.0)
      o = jnp.dot(p, wp2_ref[...], preferred_element_type=jnp.float32)
      out_ref[...] = o + bp2_ref[...]

  return pl.pallas_call(
      body,
      grid=(nb,),
      in_specs=[
          pl.BlockSpec((_BN, _D_H), lambda i: (i, 0)),
          pl.BlockSpec((_BN, 1), lambda i: (i, 0)),
          pl.BlockSpec((1, _G), lambda i: (0, 0)),
          pl.BlockSpec((_D_H, 1), lambda i: (0, 0)),
          pl.BlockSpec((1, 1), lambda i: (0, 0)),
          pl.BlockSpec((_D_H, 128), lambda i: (0, 0)),
          pl.BlockSpec((1, 128), lambda i: (0, 0)),
          pl.BlockSpec((128, 1), lambda i: (0, 0)),
          pl.BlockSpec((1, 1), lambda i: (0, 0)),
      ],
      out_specs=pl.BlockSpec((_G, 1), lambda i: (0, 0)),
      out_shape=jax.ShapeDtypeStruct((_G, 1), jnp.float32),
      scratch_shapes=[
          pltpu.VMEM((_G, _D_H), jnp.float32),
          pltpu.VMEM((_G, 1), jnp.float32),
      ],
  )(h2, batch2, gmax, W_gate, b_gate2, W_p1, b_p12, W_p2, b_p22)


def kernel(x, edge_index, edge_attr, batch, W_emb, b_emb, W_self, W_nbr,
           W_edge, b_conv, W_gate, b_gate, W_p1, b_p1, W_p2, b_p2):
  src = edge_index[0]
  dst = edge_index[1]
  zacc = jnp.zeros((_ACC_R, 128), jnp.float32)
  batch2 = batch.reshape(_N, 1)

  h_split, hw = _tc_emb(x, W_emb, b_emb.reshape(1, _D_H), W_nbr)
  eaw = _tc_eaw(edge_attr, W_edge)
  agg4 = _sc_agg(hw.reshape(2 * _N, 128), eaw.reshape(2 * _E, 128), src, dst,
                 zacc)
  AGG = (agg4[:, :, :_SLAB, :].reshape(_NC, _NS * _SLAB, 128)[:, :_N, :]
         .transpose(1, 0, 2).reshape(_N, _D_H))
  h2, gmax = _tc_conv(h_split, AGG, batch2, W_self, b_conv.reshape(1, _D_H),
                      W_gate, b_gate.reshape(1, 1))
  out = _tc_pool(h2, batch2, gmax, W_gate, b_gate.reshape(1, 1), W_p1,
                 b_p1.reshape(1, 128), W_p2, b_p2.reshape(1, 1))
  return out[:, 0]


# speedup vs baseline: 1.3878x; 1.0876x over previous
import functools

import jax
import jax.numpy as jnp
from jax import lax
from jax.experimental import pallas as pl
from jax.experimental.pallas import tpu as pltpu
from jax.experimental.pallas import tpu_sc as plsc

_N = 10000
_E = 320000
_D_IN = 128
_D_H = 256
_D_EDGE = 16
_G = 64

_NC = 2
_NS = 16
_L = 16

_SLAB = 632
_ACC_R = 640
_DUMP = _SLAB
_SCAN = 2560
_NSC = _E // _SCAN
_F = 128

_BN = 1000
_BE = 4000


def _sc_agg(hw2, ea2, edge_index, zacc):
  mesh = plsc.VectorSubcoreMesh(core_axis_name="c", subcore_axis_name="s",
                                num_cores=_NC, num_subcores=_NS)

  @functools.partial(
      pl.kernel,
      mesh=mesh,
      compiler_params=pltpu.CompilerParams(needs_layout_passes=False),
      out_type=jax.ShapeDtypeStruct((_NC, _NS, _ACC_R, 128), jnp.float32),
      scratch_types=[
          pltpu.VMEM((_ACC_R, 128), jnp.float32),
          pltpu.VMEM((2, 2, _SCAN), jnp.int32),
          pltpu.VMEM((2 * _F,), jnp.int32),
          pltpu.VMEM((2 * _F,), jnp.int32),
          pltpu.VMEM((2 * _F,), jnp.int32),
          pltpu.VMEM((_F, 128), jnp.float32),
          pltpu.VMEM((_F, 128), jnp.float32),
          pltpu.SemaphoreType.DMA,
          pltpu.SemaphoreType.DMA,
          pltpu.SemaphoreType.DMA,
      ],
  )
  def seg_kernel(hw_hbm, ea_hbm, ei_hbm, z_hbm, out_hbm, acc,
                 sdb, gbuf, ebuf, lbuf, rows, earows, sem, sem2, sems):
    c = lax.axis_index("c")
    s = lax.axis_index("s")
    lo = s * _SLAB
    goff = c * _N
    eoff = c * _E
    iot = lax.iota(jnp.int32, _L)
    pltpu.sync_copy(z_hbm, acc)

    def flush(off):
      pltpu.async_copy(hw_hbm.at[gbuf.at[pl.ds(off, _F)]], rows, sem)
      pltpu.async_copy(ea_hbm.at[ebuf.at[pl.ds(off, _F)]], earows, sem2)
      pltpu.make_async_copy(hw_hbm.at[gbuf.at[pl.ds(off, _F)]], rows,
                            sem).wait()
      pltpu.make_async_copy(ea_hbm.at[ebuf.at[pl.ds(off, _F)]], earows,
                            sem2).wait()
      def group(k, carry):
        lv = lbuf[pl.ds(off + k * _L, _L)]
        for t in range(_L):
          d = lv[t]
          for q in range(8):
            sl = pl.ds(q * _L, _L)
            plsc.addupdate(acc.at[d, sl],
                           rows[k * _L + t, sl] + earows[k * _L + t, sl])
        return carry

      lax.fori_loop(0, _F // _L, group, 0, unroll=False)

    def scan_chunk(j, cursor):
      jm = lax.rem(j, 2)
      pltpu.make_async_copy(ei_hbm.at[:, pl.ds(0, _SCAN)], sdb.at[jm],
                            sems).wait()

      @pl.when(j + 1 < _NSC)
      def _():
        pltpu.async_copy(ei_hbm.at[:, pl.ds((j + 1) * _SCAN, _SCAN)],
                         sdb.at[1 - jm], sems)

      def step(i, cursor):
        dv = sdb[jm, 1, pl.ds(i * _L, _L)]
        sv = sdb[jm, 0, pl.ds(i * _L, _L)]
        m = (dv >= lo) & (dv < lo + _SLAB)
        eid = iot + (j * _SCAN + i * _L + eoff)
        mi = jnp.where(m, 1, 0)
        ps = plsc.cumsum(mi)
        pos = (cursor + ps - 1) * mi + (2 * _F - 1) * (1 - mi)
        plsc.store_scatter(gbuf, [pos], sv + goff)
        plsc.store_scatter(ebuf, [pos], eid)
        plsc.store_scatter(lbuf, [pos], dv - lo)
        cursor = cursor + ps[_L - 1]

        @pl.when(cursor >= _F)
        def _():
          flush(0)
          gt = gbuf[pl.ds(_F, _L)]
          et = ebuf[pl.ds(_F, _L)]
          lt = lbuf[pl.ds(_F, _L)]
          gbuf[pl.ds(0, _L)] = gt
          ebuf[pl.ds(0, _L)] = et
          lbuf[pl.ds(0, _L)] = lt

        cursor = jnp.where(cursor >= _F, cursor - _F, cursor)
        return cursor

      return lax.fori_loop(0, _SCAN // _L, step, cursor, unroll=False)

    pltpu.async_copy(ei_hbm.at[:, pl.ds(0, _SCAN)], sdb.at[0], sems)
    cursor = lax.fori_loop(0, _NSC, scan_chunk, jnp.int32(0), unroll=False)

    for k in range(_F // _L):
      sl = pl.ds(k * _L, _L)
      keep = (iot + (k * _L)) < cursor
      gbuf[sl] = jnp.where(keep, gbuf[sl], goff)
      ebuf[sl] = jnp.where(keep, ebuf[sl], eoff)
      lbuf[sl] = jnp.where(keep, lbuf[sl], _DUMP)
    flush(0)

    pltpu.sync_copy(acc, out_hbm.at[c, s])

  return seg_kernel(hw2, ea2, edge_index, zacc)


def _tc_emb(x, W_emb, b_emb2, W_nbr):
  nb = _N // _BN

  def body(x_ref, w_ref, b_ref, wn_ref, h_ref, hw_ref):
    h = jnp.dot(x_ref[...], w_ref[...], preferred_element_type=jnp.float32)
    h = jnp.maximum(h + b_ref[...], 0.0)
    h_ref[0] = h[:, :128]
    h_ref[1] = h[:, 128:]
    hw = jnp.dot(h, wn_ref[...], preferred_element_type=jnp.float32)
    hw_ref[0] = hw[:, :128]
    hw_ref[1] = hw[:, 128:]

  return pl.pallas_call(
      body,
      grid=(nb,),
      in_specs=[
          pl.BlockSpec((_BN, _D_IN), lambda i: (i, 0)),
          pl.BlockSpec((_D_IN, _D_H), lambda i: (0, 0)),
          pl.BlockSpec((1, _D_H), lambda i: (0, 0)),
          pl.BlockSpec((_D_H, _D_H), lambda i: (0, 0)),
      ],
      out_specs=[
          pl.BlockSpec((2, _BN, 128), lambda i: (0, i, 0)),
          pl.BlockSpec((2, _BN, 128), lambda i: (0, i, 0)),
      ],
      out_shape=[
          jax.ShapeDtypeStruct((2, _N, 128), jnp.float32),
          jax.ShapeDtypeStruct((2, _N, 128), jnp.float32),
      ],
  )(x, W_emb, b_emb2, W_nbr)


def _tc_eaw(edge_attr, W_edge):
  nb = _E // _BE

  def body(ea_ref, we_ref, out_ref):
    eaw = jnp.dot(ea_ref[...], we_ref[...], preferred_element_type=jnp.float32)
    out_ref[0] = eaw[:, :128]
    out_ref[1] = eaw[:, 128:]

  return pl.pallas_call(
      body,
      grid=(nb,),
      in_specs=[
          pl.BlockSpec((_BE, _D_EDGE), lambda i: (i, 0)),
          pl.BlockSpec((_D_EDGE, _D_H), lambda i: (0, 0)),
      ],
      out_specs=pl.BlockSpec((2, _BE, 128), lambda i: (0, i, 0)),
      out_shape=jax.ShapeDtypeStruct((2, _E, 128), jnp.float32),
  )(edge_attr, W_edge)


def _tc_conv(h_split, AGG, batch2, W_self, b_conv2, W_gate, b_gate2):
  nb = _N // _BN

  def body(h_ref, agg_ref, b_ref, ws_ref, bc_ref, wg_ref, bg_ref, h2_ref,
           gmax_ref):
    i = pl.program_id(0)
    z = jnp.dot(h_ref[0], ws_ref[:128], preferred_element_type=jnp.float32)
    z += jnp.dot(h_ref[1], ws_ref[128:], preferred_element_type=jnp.float32)
    h2 = jnp.maximum(z + agg_ref[...] + bc_ref[...], 0.0)
    h2_ref[...] = h2
    gate = jnp.dot(h2, wg_ref[...], preferred_element_type=jnp.float32)
    gate += bg_ref[...]
    mask = lax.broadcasted_iota(jnp.int32, (_BN, _G), 1) == b_ref[...]
    gm = jnp.max(jnp.where(mask, gate, -jnp.inf), axis=0, keepdims=True)

    @pl.when(i == 0)
    def _():
      gmax_ref[...] = gm

    @pl.when(i > 0)
    def _():
      gmax_ref[...] = jnp.maximum(gmax_ref[...], gm)

  return pl.pallas_call(
      body,
      grid=(nb,),
      in_specs=[
          pl.BlockSpec((2, _BN, 128), lambda i: (0, i, 0)),
          pl.BlockSpec((_BN, _D_H), lambda i: (i, 0)),
          pl.BlockSpec((_BN, 1), lambda i: (i, 0)),
          pl.BlockSpec((_D_H, _D_H), lambda i: (0, 0)),
          pl.BlockSpec((1, _D_H), lambda i: (0, 0)),
          pl.BlockSpec((_D_H, 1), lambda i: (0, 0)),
          pl.BlockSpec((1, 1), lambda i: (0, 0)),
      ],
      out_specs=[
          pl.BlockSpec((_BN, _D_H), lambda i: (i, 0)),
          pl.BlockSpec((1, _G), lambda i: (0, 0)),
      ],
      out_shape=[
          jax.ShapeDtypeStruct((_N, _D_H), jnp.float32),
          jax.ShapeDtypeStruct((1, _G), jnp.float32),
      ],
  )(h_split, AGG, batch2, W_self, b_conv2, W_gate, b_gate2)


def _tc_pool(h2, batch2, gmax, W_gate, b_gate2, W_p1, b_p12, W_p2, b_p22):
  nb = _N // _BN

  def body(h2_ref, b_ref, gm_ref, wg_ref, bg_ref, wp1_ref, bp1_ref, wp2_ref,
           bp2_ref, out_ref, up_acc, den_acc):
    i = pl.program_id(0)

    @pl.when(i == 0)
    def _():
      up_acc[...] = jnp.zeros_like(up_acc)
      den_acc[...] = jnp.zeros_like(den_acc)

    h2 = h2_ref[...]
    gate = jnp.dot(h2, wg_ref[...], preferred_element_type=jnp.float32)
    gate += bg_ref[...]
    mask = lax.broadcasted_iota(jnp.int32, (_BN, _G), 1) == b_ref[...]
    gm_row = jnp.sum(jnp.where(mask, gm_ref[...], 0.0), axis=1, keepdims=True)
    e = jnp.exp(gate - gm_row)
    we = jnp.where(mask, e, 0.0)
    den_acc[...] += lax.dot_general(we, jnp.ones((_BN, 1), jnp.float32),
                                    (((0,), (0,)), ((), ())),
                                    preferred_element_type=jnp.float32)
    up_acc[...] += lax.dot_general(we, h2, (((0,), (0,)), ((), ())),
                                   preferred_element_type=jnp.float32)

    @pl.when(i == nb - 1)
    def _():
      den = den_acc[...]
      pooled = up_acc[...] * jnp.where(den > 0.5, 1.0 / den, 0.0)
      p = jnp.dot(pooled, wp1_ref[...], preferred_element_type=jnp.float32)
      p = jnp.maximum(p + bp1_ref[...], 0.0)
      o = jnp.dot(p, wp2_ref[...], preferred_element_type=jnp.float32)
      out_ref[...] = o + bp2_ref[...]

  return pl.pallas_call(
      body,
      grid=(nb,),
      in_specs=[
          pl.BlockSpec((_BN, _D_H), lambda i: (i, 0)),
          pl.BlockSpec((_BN, 1), lambda i: (i, 0)),
          pl.BlockSpec((1, _G), lambda i: (0, 0)),
          pl.BlockSpec((_D_H, 1), lambda i: (0, 0)),
          pl.BlockSpec((1, 1), lambda i: (0, 0)),
          pl.BlockSpec((_D_H, 128), lambda i: (0, 0)),
          pl.BlockSpec((1, 128), lambda i: (0, 0)),
          pl.BlockSpec((128, 1), lambda i: (0, 0)),
          pl.BlockSpec((1, 1), lambda i: (0, 0)),
      ],
      out_specs=pl.BlockSpec((_G, 1), lambda i: (0, 0)),
      out_shape=jax.ShapeDtypeStruct((_G, 1), jnp.float32),
      scratch_shapes=[
          pltpu.VMEM((_G, _D_H), jnp.float32),
          pltpu.VMEM((_G, 1), jnp.float32),
      ],
  )(h2, batch2, gmax, W_gate, b_gate2, W_p1, b_p12, W_p2, b_p22)


def kernel(x, edge_index, edge_attr, batch, W_emb, b_emb, W_self, W_nbr,
           W_edge, b_conv, W_gate, b_gate, W_p1, b_p1, W_p2, b_p2):
  src = edge_index[0]
  dst = edge_index[1]
  zacc = jnp.zeros((_ACC_R, 128), jnp.float32)
  batch2 = batch.reshape(_N, 1)

  h_split, hw = _tc_emb(x, W_emb, b_emb.reshape(1, _D_H), W_nbr)
  eaw = _tc_eaw(edge_attr, W_edge)
  agg4 = _sc_agg(hw.reshape(2 * _N, 128), eaw.reshape(2 * _E, 128),
                 edge_index, zacc)
  AGG = (agg4[:, :, :_SLAB, :].reshape(_NC, _NS * _SLAB, 128)[:, :_N, :]
         .transpose(1, 0, 2).reshape(_N, _D_H))
  h2, gmax = _tc_conv(h_split, AGG, batch2, W_self, b_conv.reshape(1, _D_H),
                      W_gate, b_gate.reshape(1, 1))
  out = _tc_pool(h2, batch2, gmax, W_gate, b_gate.reshape(1, 1), W_p1,
                 b_p1.reshape(1, 128), W_p2, b_p2.reshape(1, 1))
  return out[:, 0]


# speedup vs baseline: 1.6008x; 1.1535x over previous
import functools

import jax
import jax.numpy as jnp
from jax import lax
from jax.experimental import pallas as pl
from jax.experimental.pallas import tpu as pltpu
from jax.experimental.pallas import tpu_sc as plsc

_N = 10000
_E = 320000
_D_IN = 128
_D_H = 256
_D_EDGE = 16
_G = 64

_NC = 2
_NS = 16
_L = 16

_SLAB = 632
_ACC_R = 640
_DUMP = _SLAB
_SCAN = 2560
_NSC = _E // _SCAN
_F = 128

_BN = 1000
_BE = 4000


def _sc_agg(hw2, ea2, edge_index, zacc):
  mesh = plsc.VectorSubcoreMesh(core_axis_name="c", subcore_axis_name="s",
                                num_cores=_NC, num_subcores=_NS)

  @functools.partial(
      pl.kernel,
      mesh=mesh,
      compiler_params=pltpu.CompilerParams(needs_layout_passes=False),
      out_type=jax.ShapeDtypeStruct((_NC, _NS, _ACC_R, 128), jnp.float32),
      scratch_types=[
          pltpu.VMEM((_ACC_R, 128), jnp.float32),
          pltpu.VMEM((2, 2, _SCAN), jnp.int32),
          pltpu.VMEM((4 * _F,), jnp.int32),
          pltpu.VMEM((4 * _F,), jnp.int32),
          pltpu.VMEM((4 * _F,), jnp.int32),
          pltpu.VMEM((_F, 128), jnp.float32),
          pltpu.VMEM((_F, 128), jnp.float32),
          pltpu.SemaphoreType.DMA,
          pltpu.SemaphoreType.DMA,
          pltpu.SemaphoreType.DMA,
      ],
  )
  def seg_kernel(hw_hbm, ea_hbm, ei_hbm, z_hbm, out_hbm, acc,
                 sdb, gbuf, ebuf, lbuf, rows, earows, sem, sem2, sems):
    c = lax.axis_index("c")
    s = lax.axis_index("s")
    lo = s * _SLAB
    goff = c * _N
    eoff = c * _E
    iot = lax.iota(jnp.int32, _L)
    pltpu.sync_copy(z_hbm, acc)

    def issue(sd):
      b = sd * (2 * _F)
      pltpu.async_copy(hw_hbm.at[gbuf.at[pl.ds(b, _F)]], rows, sem)
      pltpu.async_copy(ea_hbm.at[ebuf.at[pl.ds(b, _F)]], earows, sem2)

    def wait_gathers(sd):
      b = sd * (2 * _F)
      pltpu.make_async_copy(hw_hbm.at[gbuf.at[pl.ds(b, _F)]], rows,
                            sem).wait()
      pltpu.make_async_copy(ea_hbm.at[ebuf.at[pl.ds(b, _F)]], earows,
                            sem2).wait()

    def accumulate(sd):
      b = sd * (2 * _F)

      def group(k, carry):
        lv = lbuf[pl.ds(b + k * _L, _L)]
        for t in range(_L):
          d = lv[t]
          for q in range(8):
            sl = pl.ds(q * _L, _L)
            plsc.addupdate(acc.at[d, sl],
                           rows[k * _L + t, sl] + earows[k * _L + t, sl])
        return carry

      lax.fori_loop(0, _F // _L, group, 0, unroll=False)

    def scan_chunk(j, carry):
      cursor, side, infl = carry
      jm = lax.rem(j, 2)
      pltpu.make_async_copy(ei_hbm.at[:, pl.ds(0, _SCAN)], sdb.at[jm],
                            sems).wait()

      @pl.when(j + 1 < _NSC)
      def _():
        pltpu.async_copy(ei_hbm.at[:, pl.ds((j + 1) * _SCAN, _SCAN)],
                         sdb.at[1 - jm], sems)

      def step(i, carry):
        cursor, side, infl = carry
        dv = sdb[jm, 1, pl.ds(i * _L, _L)]
        sv = sdb[jm, 0, pl.ds(i * _L, _L)]
        m = (dv >= lo) & (dv < lo + _SLAB)
        eid = iot + (j * _SCAN + i * _L + eoff)
        mi = jnp.where(m, 1, 0)
        ps = plsc.cumsum(mi)
        sb = side * (2 * _F)
        pos = sb + (cursor + ps - 1) * mi + (2 * _F - 1) * (1 - mi)
        plsc.store_scatter(gbuf, [pos], sv + goff)
        plsc.store_scatter(ebuf, [pos], eid)
        plsc.store_scatter(lbuf, [pos], dv - lo)
        cursor = cursor + ps[_L - 1]

        @pl.when(cursor >= _F)
        def _():
          @pl.when(infl == 1)
          def _():
            wait_gathers(1 - side)
            accumulate(1 - side)

          issue(side)
          ob = (1 - side) * (2 * _F)
          gbuf[pl.ds(ob, _L)] = gbuf[pl.ds(sb + _F, _L)]
          ebuf[pl.ds(ob, _L)] = ebuf[pl.ds(sb + _F, _L)]
          lbuf[pl.ds(ob, _L)] = lbuf[pl.ds(sb + _F, _L)]

        flip = cursor >= _F
        cursor = jnp.where(flip, cursor - _F, cursor)
        side = jnp.where(flip, 1 - side, side)
        infl = jnp.where(flip, 1, infl)
        return (cursor, side, infl)

      return lax.fori_loop(0, _SCAN // _L, step, (cursor, side, infl),
                           unroll=False)

    pltpu.async_copy(ei_hbm.at[:, pl.ds(0, _SCAN)], sdb.at[0], sems)
    cursor, side, infl = lax.fori_loop(
        0, _NSC, scan_chunk, (jnp.int32(0), jnp.int32(0), jnp.int32(0)),
        unroll=False)

    @pl.when(infl == 1)
    def _():
      wait_gathers(1 - side)
      accumulate(1 - side)

    fb = side * (2 * _F)
    for k in range(_F // _L):
      sl = pl.ds(fb + k * _L, _L)
      keep = (iot + (k * _L)) < cursor
      gbuf[sl] = jnp.where(keep, gbuf[sl], goff)
      ebuf[sl] = jnp.where(keep, ebuf[sl], eoff)
      lbuf[sl] = jnp.where(keep, lbuf[sl], _DUMP)
    issue(side)
    wait_gathers(side)
    accumulate(side)

    pltpu.sync_copy(acc, out_hbm.at[c, s])

  return seg_kernel(hw2, ea2, edge_index, zacc)


def _tc_emb(x, W_emb, b_emb2, W_nbr):
  nb = _N // _BN

  def body(x_ref, w_ref, b_ref, wn_ref, h_ref, hw_ref):
    h = jnp.dot(x_ref[...], w_ref[...], preferred_element_type=jnp.float32)
    h = jnp.maximum(h + b_ref[...], 0.0)
    h_ref[0] = h[:, :128]
    h_ref[1] = h[:, 128:]
    hw = jnp.dot(h, wn_ref[...], preferred_element_type=jnp.float32)
    hw_ref[0] = hw[:, :128]
    hw_ref[1] = hw[:, 128:]

  return pl.pallas_call(
      body,
      grid=(nb,),
      in_specs=[
          pl.BlockSpec((_BN, _D_IN), lambda i: (i, 0)),
          pl.BlockSpec((_D_IN, _D_H), lambda i: (0, 0)),
          pl.BlockSpec((1, _D_H), lambda i: (0, 0)),
          pl.BlockSpec((_D_H, _D_H), lambda i: (0, 0)),
      ],
      out_specs=[
          pl.BlockSpec((2, _BN, 128), lambda i: (0, i, 0)),
          pl.BlockSpec((2, _BN, 128), lambda i: (0, i, 0)),
      ],
      out_shape=[
          jax.ShapeDtypeStruct((2, _N, 128), jnp.float32),
          jax.ShapeDtypeStruct((2, _N, 128), jnp.float32),
      ],
  )(x, W_emb, b_emb2, W_nbr)


def _tc_eaw(edge_attr, W_edge):
  nb = _E // _BE

  def body(ea_ref, we_ref, out_ref):
    eaw = jnp.dot(ea_ref[...], we_ref[...], preferred_element_type=jnp.float32)
    out_ref[0] = eaw[:, :128]
    out_ref[1] = eaw[:, 128:]

  return pl.pallas_call(
      body,
      grid=(nb,),
      in_specs=[
          pl.BlockSpec((_BE, _D_EDGE), lambda i: (i, 0)),
          pl.BlockSpec((_D_EDGE, _D_H), lambda i: (0, 0)),
      ],
      out_specs=pl.BlockSpec((2, _BE, 128), lambda i: (0, i, 0)),
      out_shape=jax.ShapeDtypeStruct((2, _E, 128), jnp.float32),
  )(edge_attr, W_edge)


def _tc_conv(h_split, AGG, batch2, W_self, b_conv2, W_gate, b_gate2):
  nb = _N // _BN

  def body(h_ref, agg_ref, b_ref, ws_ref, bc_ref, wg_ref, bg_ref, h2_ref,
           gmax_ref):
    i = pl.program_id(0)
    z = jnp.dot(h_ref[0], ws_ref[:128], preferred_element_type=jnp.float32)
    z += jnp.dot(h_ref[1], ws_ref[128:], preferred_element_type=jnp.float32)
    h2 = jnp.maximum(z + agg_ref[...] + bc_ref[...], 0.0)
    h2_ref[...] = h2
    gate = jnp.dot(h2, wg_ref[...], preferred_element_type=jnp.float32)
    gate += bg_ref[...]
    mask = lax.broadcasted_iota(jnp.int32, (_BN, _G), 1) == b_ref[...]
    gm = jnp.max(jnp.where(mask, gate, -jnp.inf), axis=0, keepdims=True)

    @pl.when(i == 0)
    def _():
      gmax_ref[...] = gm

    @pl.when(i > 0)
    def _():
      gmax_ref[...] = jnp.maximum(gmax_ref[...], gm)

  return pl.pallas_call(
      body,
      grid=(nb,),
      in_specs=[
          pl.BlockSpec((2, _BN, 128), lambda i: (0, i, 0)),
          pl.BlockSpec((_BN, _D_H), lambda i: (i, 0)),
          pl.BlockSpec((_BN, 1), lambda i: (i, 0)),
          pl.BlockSpec((_D_H, _D_H), lambda i: (0, 0)),
          pl.BlockSpec((1, _D_H), lambda i: (0, 0)),
          pl.BlockSpec((_D_H, 1), lambda i: (0, 0)),
          pl.BlockSpec((1, 1), lambda i: (0, 0)),
      ],
      out_specs=[
          pl.BlockSpec((_BN, _D_H), lambda i: (i, 0)),
          pl.BlockSpec((1, _G), lambda i: (0, 0)),
      ],
      out_shape=[
          jax.ShapeDtypeStruct((_N, _D_H), jnp.float32),
          jax.ShapeDtypeStruct((1, _G), jnp.float32),
      ],
  )(h_split, AGG, batch2, W_self, b_conv2, W_gate, b_gate2)


def _tc_pool(h2, batch2, gmax, W_gate, b_gate2, W_p1, b_p12, W_p2, b_p22):
  nb = _N // _BN

  def body(h2_ref, b_ref, gm_ref, wg_ref, bg_ref, wp1_ref, bp1_ref, wp2_ref,
           bp2_ref, out_ref, up_acc, den_acc):
    i = pl.program_id(0)

    @pl.when(i == 0)
    def _():
      up_acc[...] = jnp.zeros_like(up_acc)
      den_acc[...] = jnp.zeros_like(den_acc)

    h2 = h2_ref[...]
    gate = jnp.dot(h2, wg_ref[...], preferred_element_type=jnp.float32)
    gate += bg_ref[...]
    mask = lax.broadcasted_iota(jnp.int32, (_BN, _G), 1) == b_ref[...]
    gm_row = jnp.sum(jnp.where(mask, gm_ref[...], 0.0), axis=1, keepdims=True)
    e = jnp.exp(gate - gm_row)
    we = jnp.where(mask, e, 0.0)
    den_acc[...] += lax.dot_general(we, jnp.ones((_BN, 1), jnp.float32),
                                    (((0,), (0,)), ((), ())),
                                    preferred_element_type=jnp.float32)
    up_acc[...] += lax.dot_general(we, h2, (((0,), (0,)), ((), ())),
                                   preferred_element_type=jnp.float32)

    @pl.when(i == nb - 1)
    def _():
      den = den_acc[...]
      pooled = up_acc[...] * jnp.where(den > 0.5, 1.0 / den, 0.0)
      p = jnp.dot(pooled, wp1_ref[...], preferred_element_type=jnp.float32)
      p = jnp.maximum(p + bp1_ref[...], 0.0)
      o = jnp.dot(p, wp2_ref[...], preferred_element_type=jnp.float32)
      out_ref[...] = o + bp2_ref[...]

  return pl.pallas_call(
      body,
      grid=(nb,),
      in_specs=[
          pl.BlockSpec((_BN, _D_H), lambda i: (i, 0)),
          pl.BlockSpec((_BN, 1), lambda i: (i, 0)),
          pl.BlockSpec((1, _G), lambda i: (0, 0)),
          pl.BlockSpec((_D_H, 1), lambda i: (0, 0)),
          pl.BlockSpec((1, 1), lambda i: (0, 0)),
          pl.BlockSpec((_D_H, 128), lambda i: (0, 0)),
          pl.BlockSpec((1, 128), lambda i: (0, 0)),
          pl.BlockSpec((128, 1), lambda i: (0, 0)),
          pl.BlockSpec((1, 1), lambda i: (0, 0)),
      ],
      out_specs=pl.BlockSpec((_G, 1), lambda i: (0, 0)),
      out_shape=jax.ShapeDtypeStruct((_G, 1), jnp.float32),
      scratch_shapes=[
          pltpu.VMEM((_G, _D_H), jnp.float32),
          pltpu.VMEM((_G, 1), jnp.float32),
      ],
  )(h2, batch2, gmax, W_gate, b_gate2, W_p1, b_p12, W_p2, b_p22)


def kernel(x, edge_index, edge_attr, batch, W_emb, b_emb, W_self, W_nbr,
           W_edge, b_conv, W_gate, b_gate, W_p1, b_p1, W_p2, b_p2):
  src = edge_index[0]
  dst = edge_index[1]
  zacc = jnp.zeros((_ACC_R, 128), jnp.float32)
  batch2 = batch.reshape(_N, 1)

  h_split, hw = _tc_emb(x, W_emb, b_emb.reshape(1, _D_H), W_nbr)
  eaw = _tc_eaw(edge_attr, W_edge)
  agg4 = _sc_agg(hw.reshape(2 * _N, 128), eaw.reshape(2 * _E, 128),
                 edge_index, zacc)
  AGG = (agg4[:, :, :_SLAB, :].reshape(_NC, _NS * _SLAB, 128)[:, :_N, :]
         .transpose(1, 0, 2).reshape(_N, _D_H))
  h2, gmax = _tc_conv(h_split, AGG, batch2, W_self, b_conv.reshape(1, _D_H),
                      W_gate, b_gate.reshape(1, 1))
  out = _tc_pool(h2, batch2, gmax, W_gate, b_gate.reshape(1, 1), W_p1,
                 b_p1.reshape(1, 128), W_p2, b_p2.reshape(1, 1))
  return out[:, 0]


# speedup vs baseline: 1.7243x; 1.0771x over previous
import functools

import jax
import jax.numpy as jnp
from jax import lax
from jax.experimental import pallas as pl
from jax.experimental.pallas import tpu as pltpu
from jax.experimental.pallas import tpu_sc as plsc

_N = 10000
_E = 320000
_D_IN = 128
_D_H = 256
_D_EDGE = 16
_G = 64

_NC = 2
_NS = 16
_L = 16

_SLAB = 632
_ACC_R = 640
_DUMP = _SLAB
_SCAN = 2560
_NSC = _E // _SCAN
_F = 128

_BN = 1000
_BE = 4000


def _sc_agg(hw2, ea2, edge_index, zacc):
  mesh = plsc.VectorSubcoreMesh(core_axis_name="c", subcore_axis_name="s",
                                num_cores=_NC, num_subcores=_NS)

  @functools.partial(
      pl.kernel,
      mesh=mesh,
      compiler_params=pltpu.CompilerParams(needs_layout_passes=False),
      out_type=jax.ShapeDtypeStruct((_NC, _NS, _ACC_R, 128), jnp.float32),
      scratch_types=[
          pltpu.VMEM((_ACC_R, 128), jnp.float32),
          pltpu.VMEM((2, 2, _SCAN), jnp.int32),
          pltpu.VMEM((4 * _F,), jnp.int32),
          pltpu.VMEM((4 * _F,), jnp.int32),
          pltpu.VMEM((4 * _F,), jnp.int32),
          pltpu.VMEM((_F, 128), jnp.float32),
          pltpu.VMEM((_F, 128), jnp.float32),
          pltpu.SemaphoreType.DMA,
          pltpu.SemaphoreType.DMA,
          pltpu.SemaphoreType.DMA,
      ],
  )
  def seg_kernel(hw_hbm, ea_hbm, ei_hbm, z_hbm, out_hbm, acc,
                 sdb, gbuf, ebuf, lbuf, rows, earows, sem, sem2, sems):
    c = lax.axis_index("c")
    s = lax.axis_index("s")
    lo = s * _SLAB
    goff = c * _N
    eoff = c * _E
    iot = lax.iota(jnp.int32, _L)
    pltpu.sync_copy(z_hbm, acc)

    def issue(sd):
      b = sd * (2 * _F)
      pltpu.async_copy(hw_hbm.at[gbuf.at[pl.ds(b, _F)]], rows, sem)
      pltpu.async_copy(ea_hbm.at[ebuf.at[pl.ds(b, _F)]], earows, sem2)

    def wait_gathers(sd):
      b = sd * (2 * _F)
      pltpu.make_async_copy(hw_hbm.at[gbuf.at[pl.ds(b, _F)]], rows,
                            sem).wait()
      pltpu.make_async_copy(ea_hbm.at[ebuf.at[pl.ds(b, _F)]], earows,
                            sem2).wait()

    def accumulate(sd):
      b = sd * (2 * _F)

      def group(k, carry):
        lv = lbuf[pl.ds(b + k * _L, _L)]
        for t in range(_L):
          d = lv[t]
          for q in range(8):
            sl = pl.ds(q * _L, _L)
            plsc.addupdate(acc.at[d, sl],
                           rows[k * _L + t, sl] + earows[k * _L + t, sl])
        return carry

      lax.fori_loop(0, _F // _L, group, 0, unroll=False)

    def scan_chunk(j, carry):
      cursor, side, infl = carry
      jm = lax.rem(j, 2)
      pltpu.make_async_copy(ei_hbm.at[:, pl.ds(0, _SCAN)], sdb.at[jm],
                            sems).wait()

      @pl.when(j + 1 < _NSC)
      def _():
        pltpu.async_copy(ei_hbm.at[:, pl.ds((j + 1) * _SCAN, _SCAN)],
                         sdb.at[1 - jm], sems)

      def step(i, carry):
        cursor, side, infl = carry
        sb = side * (2 * _F)
        for g in range(2):
          dv = sdb[jm, 1, pl.ds(i * (2 * _L) + g * _L, _L)]
          sv = sdb[jm, 0, pl.ds(i * (2 * _L) + g * _L, _L)]
          m = (dv >= lo) & (dv < lo + _SLAB)
          eid = iot + (j * _SCAN + i * (2 * _L) + g * _L + eoff)
          mi = jnp.where(m, 1, 0)
          ps = plsc.cumsum(mi)
          pos = sb + (cursor + ps - 1) * mi + (2 * _F - 1) * (1 - mi)
          plsc.store_scatter(gbuf, [pos], sv + goff)
          plsc.store_scatter(ebuf, [pos], eid)
          plsc.store_scatter(lbuf, [pos], dv - lo)
          cursor = cursor + ps[_L - 1]

        @pl.when(cursor >= _F)
        def _():
          @pl.when(infl == 1)
          def _():
            wait_gathers(1 - side)
            accumulate(1 - side)

          issue(side)
          ob = (1 - side) * (2 * _F)
          for w in range(2):
            gbuf[pl.ds(ob + w * _L, _L)] = gbuf[pl.ds(sb + _F + w * _L, _L)]
            ebuf[pl.ds(ob + w * _L, _L)] = ebuf[pl.ds(sb + _F + w * _L, _L)]
            lbuf[pl.ds(ob + w * _L, _L)] = lbuf[pl.ds(sb + _F + w * _L, _L)]

        flip = cursor >= _F
        cursor = jnp.where(flip, cursor - _F, cursor)
        side = jnp.where(flip, 1 - side, side)
        infl = jnp.where(flip, 1, infl)
        return (cursor, side, infl)

      return lax.fori_loop(0, _SCAN // (2 * _L), step, (cursor, side, infl),
                           unroll=False)

    pltpu.async_copy(ei_hbm.at[:, pl.ds(0, _SCAN)], sdb.at[0], sems)
    cursor, side, infl = lax.fori_loop(
        0, _NSC, scan_chunk, (jnp.int32(0), jnp.int32(0), jnp.int32(0)),
        unroll=False)

    @pl.when(infl == 1)
    def _():
      wait_gathers(1 - side)
      accumulate(1 - side)

    fb = side * (2 * _F)
    for k in range(_F // _L):
      sl = pl.ds(fb + k * _L, _L)
      keep = (iot + (k * _L)) < cursor
      gbuf[sl] = jnp.where(keep, gbuf[sl], goff)
      ebuf[sl] = jnp.where(keep, ebuf[sl], eoff)
      lbuf[sl] = jnp.where(keep, lbuf[sl], _DUMP)
    issue(side)
    wait_gathers(side)
    accumulate(side)

    pltpu.sync_copy(acc, out_hbm.at[c, s])

  return seg_kernel(hw2, ea2, edge_index, zacc)


def _tc_emb(x, W_emb, b_emb2, W_nbr):
  nb = _N // _BN

  def body(x_ref, w_ref, b_ref, wn_ref, h_ref, hw_ref):
    h = jnp.dot(x_ref[...], w_ref[...], preferred_element_type=jnp.float32)
    h = jnp.maximum(h + b_ref[...], 0.0)
    h_ref[0] = h[:, :128]
    h_ref[1] = h[:, 128:]
    hw = jnp.dot(h, wn_ref[...], preferred_element_type=jnp.float32)
    hw_ref[0] = hw[:, :128]
    hw_ref[1] = hw[:, 128:]

  return pl.pallas_call(
      body,
      grid=(nb,),
      in_specs=[
          pl.BlockSpec((_BN, _D_IN), lambda i: (i, 0)),
          pl.BlockSpec((_D_IN, _D_H), lambda i: (0, 0)),
          pl.BlockSpec((1, _D_H), lambda i: (0, 0)),
          pl.BlockSpec((_D_H, _D_H), lambda i: (0, 0)),
      ],
      out_specs=[
          pl.BlockSpec((2, _BN, 128), lambda i: (0, i, 0)),
          pl.BlockSpec((2, _BN, 128), lambda i: (0, i, 0)),
      ],
      out_shape=[
          jax.ShapeDtypeStruct((2, _N, 128), jnp.float32),
          jax.ShapeDtypeStruct((2, _N, 128), jnp.float32),
      ],
  )(x, W_emb, b_emb2, W_nbr)


def _tc_eaw(edge_attr, W_edge):
  nb = _E // _BE

  def body(ea_ref, we_ref, out_ref):
    eaw = jnp.dot(ea_ref[...], we_ref[...], preferred_element_type=jnp.float32)
    out_ref[0] = eaw[:, :128]
    out_ref[1] = eaw[:, 128:]

  return pl.pallas_call(
      body,
      grid=(nb,),
      in_specs=[
          pl.BlockSpec((_BE, _D_EDGE), lambda i: (i, 0)),
          pl.BlockSpec((_D_EDGE, _D_H), lambda i: (0, 0)),
      ],
      out_specs=pl.BlockSpec((2, _BE, 128), lambda i: (0, i, 0)),
      out_shape=jax.ShapeDtypeStruct((2, _E, 128), jnp.float32),
  )(edge_attr, W_edge)


def _tc_conv(h_split, AGG, batch2, W_self, b_conv2, W_gate, b_gate2):
  nb = _N // _BN

  def body(h_ref, agg_ref, b_ref, ws_ref, bc_ref, wg_ref, bg_ref, h2_ref,
           gmax_ref):
    i = pl.program_id(0)
    z = jnp.dot(h_ref[0], ws_ref[:128], preferred_element_type=jnp.float32)
    z += jnp.dot(h_ref[1], ws_ref[128:], preferred_element_type=jnp.float32)
    h2 = jnp.maximum(z + agg_ref[...] + bc_ref[...], 0.0)
    h2_ref[...] = h2
    gate = jnp.dot(h2, wg_ref[...], preferred_element_type=jnp.float32)
    gate += bg_ref[...]
    mask = lax.broadcasted_iota(jnp.int32, (_BN, _G), 1) == b_ref[...]
    gm = jnp.max(jnp.where(mask, gate, -jnp.inf), axis=0, keepdims=True)

    @pl.when(i == 0)
    def _():
      gmax_ref[...] = gm

    @pl.when(i > 0)
    def _():
      gmax_ref[...] = jnp.maximum(gmax_ref[...], gm)

  return pl.pallas_call(
      body,
      grid=(nb,),
      in_specs=[
          pl.BlockSpec((2, _BN, 128), lambda i: (0, i, 0)),
          pl.BlockSpec((_BN, _D_H), lambda i: (i, 0)),
          pl.BlockSpec((_BN, 1), lambda i: (i, 0)),
          pl.BlockSpec((_D_H, _D_H), lambda i: (0, 0)),
          pl.BlockSpec((1, _D_H), lambda i: (0, 0)),
          pl.BlockSpec((_D_H, 1), lambda i: (0, 0)),
          pl.BlockSpec((1, 1), lambda i: (0, 0)),
      ],
      out_specs=[
          pl.BlockSpec((_BN, _D_H), lambda i: (i, 0)),
          pl.BlockSpec((1, _G), lambda i: (0, 0)),
      ],
      out_shape=[
          jax.ShapeDtypeStruct((_N, _D_H), jnp.float32),
          jax.ShapeDtypeStruct((1, _G), jnp.float32),
      ],
  )(h_split, AGG, batch2, W_self, b_conv2, W_gate, b_gate2)


def _tc_pool(h2, batch2, gmax, W_gate, b_gate2, W_p1, b_p12, W_p2, b_p22):
  nb = _N // _BN

  def body(h2_ref, b_ref, gm_ref, wg_ref, bg_ref, wp1_ref, bp1_ref, wp2_ref,
           bp2_ref, out_ref, up_acc, den_acc):
    i = pl.program_id(0)

    @pl.when(i == 0)
    def _():
      up_acc[...] = jnp.zeros_like(up_acc)
      den_acc[...] = jnp.zeros_like(den_acc)

    h2 = h2_ref[...]
    gate = jnp.dot(h2, wg_ref[...], preferred_element_type=jnp.float32)
    gate += bg_ref[...]
    mask = lax.broadcasted_iota(jnp.int32, (_BN, _G), 1) == b_ref[...]
    gm_row = jnp.sum(jnp.where(mask, gm_ref[...], 0.0), axis=1, keepdims=True)
    e = jnp.exp(gate - gm_row)
    we = jnp.where(mask, e, 0.0)
    den_acc[...] += lax.dot_general(we, jnp.ones((_BN, 1), jnp.float32),
                                    (((0,), (0,)), ((), ())),
                                    preferred_element_type=jnp.float32)
    up_acc[...] += lax.dot_general(we, h2, (((0,), (0,)), ((), ())),
                                   preferred_element_type=jnp.float32)

    @pl.when(i == nb - 1)
    def _():
      den = den_acc[...]
      pooled = up_acc[...] * jnp.where(den > 0.5, 1.0 / den, 0.0)
      p = jnp.dot(pooled, wp1_ref[...], preferred_element_type=jnp.float32)
      p = jnp.maximum(p + bp1_ref[...], 0.0)
      o = jnp.dot(p, wp2_ref[...], preferred_element_type=jnp.float32)
      out_ref[...] = o + bp2_ref[...]

  return pl.pallas_call(
      body,
      grid=(nb,),
      in_specs=[
          pl.BlockSpec((_BN, _D_H), lambda i: (i, 0)),
          pl.BlockSpec((_BN, 1), lambda i: (i, 0)),
          pl.BlockSpec((1, _G), lambda i: (0, 0)),
          pl.BlockSpec((_D_H, 1), lambda i: (0, 0)),
          pl.BlockSpec((1, 1), lambda i: (0, 0)),
          pl.BlockSpec((_D_H, 128), lambda i: (0, 0)),
          pl.BlockSpec((1, 128), lambda i: (0, 0)),
          pl.BlockSpec((128, 1), lambda i: (0, 0)),
          pl.BlockSpec((1, 1), lambda i: (0, 0)),
      ],
      out_specs=pl.BlockSpec((_G, 1), lambda i: (0, 0)),
      out_shape=jax.ShapeDtypeStruct((_G, 1), jnp.float32),
      scratch_shapes=[
          pltpu.VMEM((_G, _D_H), jnp.float32),
          pltpu.VMEM((_G, 1), jnp.float32),
      ],
  )(h2, batch2, gmax, W_gate, b_gate2, W_p1, b_p12, W_p2, b_p22)


def kernel(x, edge_index, edge_attr, batch, W_emb, b_emb, W_self, W_nbr,
           W_edge, b_conv, W_gate, b_gate, W_p1, b_p1, W_p2, b_p2):
  src = edge_index[0]
  dst = edge_index[1]
  zacc = jnp.zeros((_ACC_R, 128), jnp.float32)
  batch2 = batch.reshape(_N, 1)

  h_split, hw = _tc_emb(x, W_emb, b_emb.reshape(1, _D_H), W_nbr)
  eaw = _tc_eaw(edge_attr, W_edge)
  agg4 = _sc_agg(hw.reshape(2 * _N, 128), eaw.reshape(2 * _E, 128),
                 edge_index, zacc)
  AGG = (agg4[:, :, :_SLAB, :].reshape(_NC, _NS * _SLAB, 128)[:, :_N, :]
         .transpose(1, 0, 2).reshape(_N, _D_H))
  h2, gmax = _tc_conv(h_split, AGG, batch2, W_self, b_conv.reshape(1, _D_H),
                      W_gate, b_gate.reshape(1, 1))
  out = _tc_pool(h2, batch2, gmax, W_gate, b_gate.reshape(1, 1), W_p1,
                 b_p1.reshape(1, 128), W_p2, b_p2.reshape(1, 1))
  return out[:, 0]
